# baseline (speedup 1.0000x reference)
import jax
import jax.numpy as jnp
from jax import lax
from jax.experimental import pallas as pl
from jax.experimental.pallas import tpu as pltpu
from jax.experimental.pallas import tpu_sc as plsc

DIM = 1024
HIDDEN = 4096
E = 8
N = 2048
CAP = 2048.0
EPS = 1e-6

T = 512
R = 128
HC = 2048
NH = HIDDEN // HC
MAX_TILES = 12
NSTEPS = MAX_TILES * NH
P = MAX_TILES * T

NC = 2
NS = 16
NW = NC * NS
TOK_W = N // NW
HTOK = TOK_W // 2


def _gating_body(x_ref, wg_ref, p_ref, w_ref, loss_ref,
                 st_t_ref, st_h_ref, st_e_ref, st_v_ref):
    x = x_ref[...]
    wg = wg_ref[...]
    logits = jnp.dot(x, wg, preferred_element_type=jnp.float32)
    m = jnp.max(logits, axis=1, keepdims=True)
    ex = jnp.exp(logits - m)
    s = ex / jnp.sum(ex, axis=1, keepdims=True)

    top = jnp.max(s, axis=1, keepdims=True)
    lane = lax.broadcasted_iota(jnp.int32, (N, E), 1)
    e_n = jnp.min(jnp.where(s == top, lane, E), axis=1, keepdims=True)
    onehot = (lane == e_n).astype(jnp.float32)

    seg = jnp.sum(s * onehot, axis=0, keepdims=True)
    counts = jnp.sum(onehot, axis=0, keepdims=True)

    imp = seg / (seg + EPS) * CAP

    def cv2(v):
        mu = jnp.mean(v)
        var = jnp.sum((v - mu) ** 2) / (E - 1)
        return var / (mu * mu + 1e-10)

    loss_ref[...] = (cv2(imp) + cv2(counts)).reshape(1, 1)

    csum = onehot
    k = 1
    while k < N:
        csum = csum + jnp.concatenate(
            [jnp.zeros((k, E), jnp.float32), csum[:N - k]], axis=0)
        k *= 2
    rank = (jnp.sum(csum * onehot, axis=1, keepdims=True) - 1.0).astype(jnp.int32)

    cnt_i = counts.astype(jnp.int32)
    nblk = (cnt_i + (T - 1)) // T
    bcum = nblk
    k = 1
    while k < E:
        bcum = bcum + jnp.concatenate(
            [jnp.zeros((1, k), jnp.int32), bcum[:, :E - k]], axis=1)
        k *= 2
    bexcl = bcum - nblk
    off_rows = (bexcl * T).astype(jnp.float32)
    off_at = jnp.sum(onehot * off_rows, axis=1, keepdims=True).astype(jnp.int32)
    p_ref[...] = off_at + rank

    seg_at = jnp.sum(onehot * seg, axis=1, keepdims=True)
    w_ref[...] = jnp.broadcast_to(top / (seg_at + EPS) * CAP, (N, 128))

    total = bcum[0, E - 1]
    e_last = jnp.sum((bcum <= total - 1).astype(jnp.int32))
    i_io = lax.broadcasted_iota(jnp.int32, (NSTEPS, 1), 0)
    active = i_io < NH * total
    t_i = jnp.where(active, i_io // NH, total - 1)
    h_i = jnp.where(active, i_io % NH, NH - 1)
    t_b = jnp.broadcast_to(t_i, (NSTEPS, E))
    bcum_b = jnp.broadcast_to(bcum, (NSTEPS, E))
    et = jnp.minimum(
        jnp.sum((bcum_b <= t_b).astype(jnp.int32), axis=1, keepdims=True),
        e_last)
    sel = (lax.broadcasted_iota(jnp.int32, (NSTEPS, E), 1) == et).astype(jnp.int32)
    bexcl_t = jnp.sum(sel * jnp.broadcast_to(bexcl, (NSTEPS, E)), axis=1, keepdims=True)
    cnt_t = jnp.sum(sel * jnp.broadcast_to(cnt_i, (NSTEPS, E)), axis=1, keepdims=True)
    k_t = t_i - bexcl_t
    v_i = jnp.where(active, jnp.clip(cnt_t - k_t * T, 0, T), 0)
    st_t_ref[...] = t_i
    st_h_ref[...] = h_i
    st_e_ref[...] = et
    st_v_ref[...] = v_i


def _run_gating(x, w_gate):
    return pl.pallas_call(
        _gating_body,
        out_shape=(
            jax.ShapeDtypeStruct((N, 1), jnp.int32),
            jax.ShapeDtypeStruct((N, 128), jnp.float32),
            jax.ShapeDtypeStruct((1, 1), jnp.float32),
            jax.ShapeDtypeStruct((NSTEPS, 1), jnp.int32),
            jax.ShapeDtypeStruct((NSTEPS, 1), jnp.int32),
            jax.ShapeDtypeStruct((NSTEPS, 1), jnp.int32),
            jax.ShapeDtypeStruct((NSTEPS, 1), jnp.int32),
        ),
    )(x, w_gate)


def _sc_mesh():
    return plsc.VectorSubcoreMesh(
        core_axis_name="c", subcore_axis_name="s", num_cores=NC, num_subcores=NS)


def _sc_scatter_body(x_hbm, p_hbm, w_hbm, xs_hbm, ws_hbm,
                     idx0, idx1, r0, r1, wb0, wb1, s0, s1, s2):
    wid = lax.axis_index("s") * NC + lax.axis_index("c")
    base = wid * TOK_W
    pltpu.sync_copy(p_hbm.at[pl.ds(base, HTOK)], idx0)
    pltpu.sync_copy(p_hbm.at[pl.ds(base + HTOK, HTOK)], idx1)
    l0 = pltpu.async_copy(x_hbm.at[pl.ds(base, HTOK)], r0, s0)
    l1 = pltpu.async_copy(x_hbm.at[pl.ds(base + HTOK, HTOK)], r1, s1)
    lw0 = pltpu.async_copy(w_hbm.at[pl.ds(base, HTOK)], wb0, s2)
    lw1 = pltpu.async_copy(w_hbm.at[pl.ds(base + HTOK, HTOK)], wb1, s2)
    l0.wait()
    st0 = pltpu.async_copy(r0, xs_hbm.at[idx0], s0)
    l1.wait()
    st1 = pltpu.async_copy(r1, xs_hbm.at[idx1], s1)
    lw0.wait()
    lw1.wait()
    sw0 = pltpu.async_copy(wb0, ws_hbm.at[idx0], s2)
    sw1 = pltpu.async_copy(wb1, ws_hbm.at[idx1], s2)
    st0.wait()
    st1.wait()
    sw0.wait()
    sw1.wait()


def _run_scatter(x, p_flat, w_flat):
    return pl.kernel(
        _sc_scatter_body,
        out_type=(
            jax.ShapeDtypeStruct((P, DIM), jnp.float32),
            jax.ShapeDtypeStruct((P, 128), jnp.float32),
        ),
        mesh=_sc_mesh(),
        scratch_types=[
            pltpu.VMEM((HTOK,), jnp.int32),
            pltpu.VMEM((HTOK,), jnp.int32),
            pltpu.VMEM((HTOK, DIM), jnp.float32),
            pltpu.VMEM((HTOK, DIM), jnp.float32),
            pltpu.VMEM((HTOK, 128), jnp.float32),
            pltpu.VMEM((HTOK, 128), jnp.float32),
            pltpu.SemaphoreType.DMA,
            pltpu.SemaphoreType.DMA,
            pltpu.SemaphoreType.DMA,
        ],
    )(x, p_flat, w_flat)


def _ffn_body(st_t, st_h, st_e, st_v, x_ref, w1_ref, b1_ref, w2_ref, b2_ref,
              ws_ref, out_ref):
    i = pl.program_id(0)
    hc = st_h[i]
    v = st_v[i]
    for r in range(T // R):
        @pl.when(r * R < v)
        def _chunk():
            xs = x_ref[r * R:(r + 1) * R, :]
            h = jnp.dot(xs, w1_ref[0], preferred_element_type=jnp.float32)
            h = jax.nn.gelu(h + b1_ref[0])
            o = jnp.dot(h, w2_ref[0], preferred_element_type=jnp.float32)
            wcol = ws_ref[r * R:(r + 1) * R, 0:1]

            @pl.when(hc == 0)
            def _init():
                acc = o + b2_ref[0]
                if NH == 1:
                    acc = acc * wcol
                out_ref[r * R:(r + 1) * R, :] = acc

            if NH > 1:
                @pl.when((hc > 0) & (hc < NH - 1))
                def _acc():
                    out_ref[r * R:(r + 1) * R, :] = (
                        out_ref[r * R:(r + 1) * R, :] + o)

                @pl.when(hc == NH - 1)
                def _fin():
                    out_ref[r * R:(r + 1) * R, :] = (
                        out_ref[r * R:(r + 1) * R, :] + o) * wcol


def _run_ffn(st_t, st_h, st_e, st_v, x_sorted, w16_sorted, W1, b1, W2, b2):
    grid_spec = pltpu.PrefetchScalarGridSpec(
        num_scalar_prefetch=4,
        grid=(NSTEPS,),
        in_specs=[
            pl.BlockSpec((T, DIM), lambda i, t, h, e, v: (t[i], 0)),
            pl.BlockSpec((1, DIM, HC), lambda i, t, h, e, v: (e[i], 0, h[i])),
            pl.BlockSpec((1, 1, HC), lambda i, t, h, e, v: (e[i], 0, h[i])),
            pl.BlockSpec((1, HC, DIM), lambda i, t, h, e, v: (e[i], h[i], 0)),
            pl.BlockSpec((1, 1, DIM), lambda i, t, h, e, v: (e[i], 0, 0)),
            pl.BlockSpec((T, 128), lambda i, t, h, e, v: (t[i], 0)),
        ],
        out_specs=pl.BlockSpec((T, DIM), lambda i, t, h, e, v: (t[i], 0)),
    )
    return pl.pallas_call(
        _ffn_body,
        grid_spec=grid_spec,
        out_shape=jax.ShapeDtypeStruct((P, DIM), jnp.float32),
        compiler_params=pltpu.CompilerParams(
            dimension_semantics=("arbitrary",)),
    )(st_t, st_h, st_e, st_v, x_sorted, W1,
      b1.reshape(E, 1, HIDDEN), W2, b2.reshape(E, 1, DIM), w16_sorted)


def _sc_gather_body(y_hbm, p_hbm, out_hbm, idx0, idx1, r0, r1, s0, s1):
    wid = lax.axis_index("s") * NC + lax.axis_index("c")
    base = wid * TOK_W
    pltpu.sync_copy(p_hbm.at[pl.ds(base, HTOK)], idx0)
    pltpu.sync_copy(p_hbm.at[pl.ds(base + HTOK, HTOK)], idx1)
    g0 = pltpu.async_copy(y_hbm.at[idx0], r0, s0)
    g1 = pltpu.async_copy(y_hbm.at[idx1], r1, s1)
    g0.wait()
    w0 = pltpu.async_copy(r0, out_hbm.at[pl.ds(base, HTOK)], s0)
    g1.wait()
    w1 = pltpu.async_copy(r1, out_hbm.at[pl.ds(base + HTOK, HTOK)], s1)
    w0.wait()
    w1.wait()


def _run_gather(y_sorted, p_flat):
    return pl.kernel(
        _sc_gather_body,
        out_type=jax.ShapeDtypeStruct((N, DIM), jnp.float32),
        mesh=_sc_mesh(),
        scratch_types=[
            pltpu.VMEM((HTOK,), jnp.int32),
            pltpu.VMEM((HTOK,), jnp.int32),
            pltpu.VMEM((HTOK, DIM), jnp.float32),
            pltpu.VMEM((HTOK, DIM), jnp.float32),
            pltpu.SemaphoreType.DMA,
            pltpu.SemaphoreType.DMA,
        ],
    )(y_sorted, p_flat)


def kernel(x, w_gate, W1, b1, W2, b2):
    p2, wtok, loss2, st_t, st_h, st_e, st_v = _run_gating(x, w_gate)
    p_flat = p2.reshape(-1)
    x_sorted, w16_sorted = _run_scatter(x, p_flat, wtok)
    y_sorted = _run_ffn(st_t.reshape(-1), st_h.reshape(-1), st_e.reshape(-1),
                        st_v.reshape(-1), x_sorted, w16_sorted, W1, b1, W2, b2)
    out = _run_gather(y_sorted, p_flat)
    return out, loss2[0, 0]


# speedup vs baseline: 1.0027x; 1.0027x over previous
import jax
import jax.numpy as jnp
from jax import lax
from jax.experimental import pallas as pl
from jax.experimental.pallas import tpu as pltpu
from jax.experimental.pallas import tpu_sc as plsc

DIM = 1024
HIDDEN = 4096
E = 8
N = 2048
CAP = 2048.0
EPS = 1e-6

T = 512
R = 128
HC = 2048
NH = HIDDEN // HC
MAX_TILES = 12
NSTEPS = MAX_TILES * NH
P = MAX_TILES * T

NC = 2
NS = 16
NW = NC * NS
TOK_W = N // NW


def _gating_body(x_ref, wg_ref, p_ref, w_ref, loss_ref,
                 st_t_ref, st_h_ref, st_e_ref, st_v_ref):
    x = x_ref[...]
    wg = wg_ref[...]
    logits = jnp.dot(x, wg, preferred_element_type=jnp.float32)
    m = jnp.max(logits, axis=1, keepdims=True)
    ex = jnp.exp(logits - m)
    s = ex / jnp.sum(ex, axis=1, keepdims=True)

    top = jnp.max(s, axis=1, keepdims=True)
    lane = lax.broadcasted_iota(jnp.int32, (N, E), 1)
    e_n = jnp.min(jnp.where(s == top, lane, E), axis=1, keepdims=True)
    onehot = (lane == e_n).astype(jnp.float32)

    seg = jnp.sum(s * onehot, axis=0, keepdims=True)
    counts = jnp.sum(onehot, axis=0, keepdims=True)

    imp = seg / (seg + EPS) * CAP

    def cv2(v):
        mu = jnp.mean(v)
        var = jnp.sum((v - mu) ** 2) / (E - 1)
        return var / (mu * mu + 1e-10)

    loss_ref[...] = (cv2(imp) + cv2(counts)).reshape(1, 1)

    csum = onehot
    k = 1
    while k < N:
        csum = csum + jnp.concatenate(
            [jnp.zeros((k, E), jnp.float32), csum[:N - k]], axis=0)
        k *= 2
    rank = (jnp.sum(csum * onehot, axis=1, keepdims=True) - 1.0).astype(jnp.int32)

    cnt_i = counts.astype(jnp.int32)
    nblk = (cnt_i + (T - 1)) // T
    bcum = nblk
    k = 1
    while k < E:
        bcum = bcum + jnp.concatenate(
            [jnp.zeros((1, k), jnp.int32), bcum[:, :E - k]], axis=1)
        k *= 2
    bexcl = bcum - nblk
    off_rows = (bexcl * T).astype(jnp.float32)
    off_at = jnp.sum(onehot * off_rows, axis=1, keepdims=True).astype(jnp.int32)
    p_ref[...] = off_at + rank

    seg_at = jnp.sum(onehot * seg, axis=1, keepdims=True)
    w_ref[...] = jnp.broadcast_to(top / (seg_at + EPS) * CAP, (N, 128))

    total = bcum[0, E - 1]
    e_last = jnp.sum((bcum <= total - 1).astype(jnp.int32))
    i_io = lax.broadcasted_iota(jnp.int32, (NSTEPS, 1), 0)
    active = i_io < NH * total
    t_i = jnp.where(active, i_io // NH, total - 1)
    h_i = jnp.where(active, i_io % NH, NH - 1)
    t_b = jnp.broadcast_to(t_i, (NSTEPS, E))
    bcum_b = jnp.broadcast_to(bcum, (NSTEPS, E))
    et = jnp.minimum(
        jnp.sum((bcum_b <= t_b).astype(jnp.int32), axis=1, keepdims=True),
        e_last)
    sel = (lax.broadcasted_iota(jnp.int32, (NSTEPS, E), 1) == et).astype(jnp.int32)
    bexcl_t = jnp.sum(sel * jnp.broadcast_to(bexcl, (NSTEPS, E)), axis=1, keepdims=True)
    cnt_t = jnp.sum(sel * jnp.broadcast_to(cnt_i, (NSTEPS, E)), axis=1, keepdims=True)
    k_t = t_i - bexcl_t
    v_i = jnp.where(active, jnp.clip(cnt_t - k_t * T, 0, T), 0)
    st_t_ref[...] = t_i
    st_h_ref[...] = h_i
    st_e_ref[...] = et
    st_v_ref[...] = v_i


def _run_gating(x, w_gate):
    return pl.pallas_call(
        _gating_body,
        out_shape=(
            jax.ShapeDtypeStruct((N, 1), jnp.int32),
            jax.ShapeDtypeStruct((N, 128), jnp.float32),
            jax.ShapeDtypeStruct((1, 1), jnp.float32),
            jax.ShapeDtypeStruct((NSTEPS, 1), jnp.int32),
            jax.ShapeDtypeStruct((NSTEPS, 1), jnp.int32),
            jax.ShapeDtypeStruct((NSTEPS, 1), jnp.int32),
            jax.ShapeDtypeStruct((NSTEPS, 1), jnp.int32),
        ),
    )(x, w_gate)


def _sc_mesh():
    return plsc.VectorSubcoreMesh(
        core_axis_name="c", subcore_axis_name="s", num_cores=NC, num_subcores=NS)


def _sc_scatter_body(x_hbm, p_hbm, w_hbm, xs_hbm, ws_hbm,
                     idx_v, rows_v, wb_v, sem):
    wid = lax.axis_index("s") * NC + lax.axis_index("c")
    base = wid * TOK_W
    pltpu.sync_copy(p_hbm.at[pl.ds(base, TOK_W)], idx_v)
    pltpu.sync_copy(x_hbm.at[pl.ds(base, TOK_W)], rows_v)
    pltpu.sync_copy(w_hbm.at[pl.ds(base, TOK_W)], wb_v)
    pltpu.async_copy(rows_v, xs_hbm.at[idx_v], sem).wait()
    pltpu.async_copy(wb_v, ws_hbm.at[idx_v], sem).wait()


def _run_scatter(x, p_flat, w_flat):
    return pl.kernel(
        _sc_scatter_body,
        out_type=(
            jax.ShapeDtypeStruct((P, DIM), jnp.float32),
            jax.ShapeDtypeStruct((P, 128), jnp.float32),
        ),
        mesh=_sc_mesh(),
        scratch_types=[
            pltpu.VMEM((TOK_W,), jnp.int32),
            pltpu.VMEM((TOK_W, DIM), jnp.float32),
            pltpu.VMEM((TOK_W, 128), jnp.float32),
            pltpu.SemaphoreType.DMA,
        ],
    )(x, p_flat, w_flat)


def _ffn_body(st_t, st_h, st_e, st_v, x_ref, w1_ref, b1_ref, w2_ref, b2_ref,
              ws_ref, out_ref):
    i = pl.program_id(0)
    hc = st_h[i]
    v = st_v[i]
    for r in range(T // R):
        @pl.when(r * R < v)
        def _chunk():
            xs = x_ref[r * R:(r + 1) * R, :]
            h = jnp.dot(xs, w1_ref[0], preferred_element_type=jnp.float32)
            h = jax.nn.gelu(h + b1_ref[0])
            o = jnp.dot(h, w2_ref[0], preferred_element_type=jnp.float32)
            wcol = ws_ref[r * R:(r + 1) * R, 0:1]

            @pl.when(hc == 0)
            def _init():
                acc = o + b2_ref[0]
                if NH == 1:
                    acc = acc * wcol
                out_ref[r * R:(r + 1) * R, :] = acc

            if NH > 1:
                @pl.when((hc > 0) & (hc < NH - 1))
                def _acc():
                    out_ref[r * R:(r + 1) * R, :] = (
                        out_ref[r * R:(r + 1) * R, :] + o)

                @pl.when(hc == NH - 1)
                def _fin():
                    out_ref[r * R:(r + 1) * R, :] = (
                        out_ref[r * R:(r + 1) * R, :] + o) * wcol


def _run_ffn(st_t, st_h, st_e, st_v, x_sorted, w16_sorted, W1, b1, W2, b2):
    grid_spec = pltpu.PrefetchScalarGridSpec(
        num_scalar_prefetch=4,
        grid=(NSTEPS,),
        in_specs=[
            pl.BlockSpec((T, DIM), lambda i, t, h, e, v: (t[i], 0)),
            pl.BlockSpec((1, DIM, HC), lambda i, t, h, e, v: (e[i], 0, h[i])),
            pl.BlockSpec((1, 1, HC), lambda i, t, h, e, v: (e[i], 0, h[i])),
            pl.BlockSpec((1, HC, DIM), lambda i, t, h, e, v: (e[i], h[i], 0)),
            pl.BlockSpec((1, 1, DIM), lambda i, t, h, e, v: (e[i], 0, 0)),
            pl.BlockSpec((T, 128), lambda i, t, h, e, v: (t[i], 0)),
        ],
        out_specs=pl.BlockSpec((T, DIM), lambda i, t, h, e, v: (t[i], 0)),
    )
    return pl.pallas_call(
        _ffn_body,
        grid_spec=grid_spec,
        out_shape=jax.ShapeDtypeStruct((P, DIM), jnp.float32),
        compiler_params=pltpu.CompilerParams(
            dimension_semantics=("arbitrary",)),
    )(st_t, st_h, st_e, st_v, x_sorted, W1,
      b1.reshape(E, 1, HIDDEN), W2, b2.reshape(E, 1, DIM), w16_sorted)


def _sc_gather_body(y_hbm, p_hbm, out_hbm, idx_v, rows_v, sem):
    wid = lax.axis_index("s") * NC + lax.axis_index("c")
    base = wid * TOK_W
    pltpu.sync_copy(p_hbm.at[pl.ds(base, TOK_W)], idx_v)
    pltpu.async_copy(y_hbm.at[idx_v], rows_v, sem).wait()
    pltpu.sync_copy(rows_v, out_hbm.at[pl.ds(base, TOK_W)])


def _run_gather(y_sorted, p_flat):
    return pl.kernel(
        _sc_gather_body,
        out_type=jax.ShapeDtypeStruct((N, DIM), jnp.float32),
        mesh=_sc_mesh(),
        scratch_types=[
            pltpu.VMEM((TOK_W,), jnp.int32),
            pltpu.VMEM((TOK_W, DIM), jnp.float32),
            pltpu.SemaphoreType.DMA,
        ],
    )(y_sorted, p_flat)


def kernel(x, w_gate, W1, b1, W2, b2):
    p2, wtok, loss2, st_t, st_h, st_e, st_v = _run_gating(x, w_gate)
    p_flat = p2.reshape(-1)
    x_sorted, w16_sorted = _run_scatter(x, p_flat, wtok)
    y_sorted = _run_ffn(st_t.reshape(-1), st_h.reshape(-1), st_e.reshape(-1),
                        st_v.reshape(-1), x_sorted, w16_sorted, W1, b1, W2, b2)
    out = _run_gather(y_sorted, p_flat)
    return out, loss2[0, 0]
